# baseline (speedup 1.0000x reference)
import functools

import jax
import jax.numpy as jnp
from jax import lax
from jax.experimental import pallas as pl
from jax.experimental.pallas import tpu as pltpu
from jax.experimental.pallas import tpu_sc as plsc

NVOCAB = 1000000
NEMBED = 64
NTOKEN = 200
BATCH = 4096

_NUM_WORKERS = 32
_BATCHES_PER_WORKER = BATCH // _NUM_WORKERS


def _emb_kernel(tokens_hbm, table_hbm, pos_hbm, out_hbm,
                idx_v, rows_v, pos_v, sem):
    nc = 2
    wid = lax.axis_index("s") * nc + lax.axis_index("c")

    pltpu.sync_copy(pos_hbm, pos_v)

    def batch_body(b, carry):
        base = (wid * _BATCHES_PER_WORKER + b) * NTOKEN
        pltpu.sync_copy(tokens_hbm.at[pl.ds(base, NTOKEN)], idx_v)
        pltpu.async_copy(table_hbm.at[idx_v], rows_v, sem).wait()

        def add_body(i, carry2):
            for j in range(NEMBED // 16):
                sl = pl.ds(j * 16, 16)
                rows_v[i, sl] += pos_v[i, sl]
            return carry2

        lax.fori_loop(0, NTOKEN, add_body, 0, unroll=False)
        pltpu.sync_copy(rows_v, out_hbm.at[pl.ds(base, NTOKEN)])
        return carry

    lax.fori_loop(0, _BATCHES_PER_WORKER, batch_body, 0, unroll=False)


@jax.jit
def kernel(tokens, token_table, pos_embed):
    tokens_flat = tokens.reshape(-1).astype(jnp.int32)
    mesh = plsc.VectorSubcoreMesh(core_axis_name="c", subcore_axis_name="s")
    out = pl.kernel(
        _emb_kernel,
        out_type=jax.ShapeDtypeStruct((BATCH * NTOKEN, NEMBED), jnp.float32),
        mesh=mesh,
        scratch_types=[
            pltpu.VMEM((NTOKEN,), jnp.int32),
            pltpu.VMEM((NTOKEN, NEMBED), jnp.float32),
            pltpu.VMEM((NTOKEN, NEMBED), jnp.float32),
            pltpu.SemaphoreType.DMA,
        ],
        compiler_params=pltpu.CompilerParams(use_tc_tiling_on_sc=False),
    )(tokens_flat, token_table, pos_embed)
    return out.reshape(BATCH, NTOKEN, NEMBED)


# speedup vs baseline: 1.2100x; 1.2100x over previous
import jax
import jax.numpy as jnp
from jax import lax
from jax.experimental import pallas as pl
from jax.experimental.pallas import tpu as pltpu
from jax.experimental.pallas import tpu_sc as plsc

NVOCAB = 1000000
NEMBED = 64
NTOKEN = 200
BATCH = 4096

_NUM_WORKERS = 32
_BPW = BATCH // _NUM_WORKERS
_NBUF = 4


def _emb_kernel(tokens_hbm, table_hbm, pos_hbm, out_hbm,
                idx_all, pos_v, rows, gsem, osem):
    nc = 2
    wid = lax.axis_index("s") * nc + lax.axis_index("c")
    tok0 = wid * _BPW * NTOKEN

    pltpu.sync_copy(tokens_hbm.at[pl.ds(tok0, _BPW * NTOKEN)], idx_all)
    pltpu.sync_copy(pos_hbm, pos_v)

    def gather_descr(b, k):
        return pltpu.make_async_copy(
            table_hbm.at[idx_all.at[pl.ds(b * NTOKEN, NTOKEN)]], rows[k], gsem)

    def scatter_descr(b, k):
        return pltpu.make_async_copy(
            rows[k], out_hbm.at[pl.ds(tok0 + b * NTOKEN, NTOKEN)], osem)

    def start(b, k, guard_scatter):
        if guard_scatter:
            @pl.when(b >= _NBUF)
            def _():
                scatter_descr(b - _NBUF, k).wait()
        gather_descr(b, k).start()

    def finish(b, k):
        gather_descr(b, k).wait()

        def add_body(i, c):
            for j in range(NEMBED // 16):
                sl = pl.ds(j * 16, 16)
                plsc.addupdate(rows[k].at[i, sl], pos_v[i, sl])
            return c

        lax.fori_loop(0, NTOKEN, add_body, 0, unroll=False)
        scatter_descr(b, k).start()

    for k in range(_NBUF - 1):
        start(k, k, guard_scatter=False)

    def loop_body(bb, c):
        for k in range(_NBUF):
            b = bb * _NBUF + k
            finish(b, k)
            nb = b + _NBUF - 1

            @pl.when(nb < _BPW)
            def _():
                start(nb, (k + _NBUF - 1) % _NBUF, guard_scatter=True)
        return c

    lax.fori_loop(0, _BPW // _NBUF, loop_body, 0, unroll=False)

    for k in range(_NBUF):
        scatter_descr(_BPW - _NBUF + k, k).wait()


@jax.jit
def kernel(tokens, token_table, pos_embed):
    tokens_flat = tokens.reshape(-1).astype(jnp.int32)
    mesh = plsc.VectorSubcoreMesh(core_axis_name="c", subcore_axis_name="s")
    out = pl.kernel(
        _emb_kernel,
        out_type=jax.ShapeDtypeStruct((BATCH * NTOKEN, NEMBED), jnp.float32),
        mesh=mesh,
        scratch_types=[
            pltpu.VMEM((_BPW * NTOKEN,), jnp.int32),
            pltpu.VMEM((NTOKEN, NEMBED), jnp.float32),
            [pltpu.VMEM((NTOKEN, NEMBED), jnp.float32) for _ in range(_NBUF)],
            pltpu.SemaphoreType.DMA,
            pltpu.SemaphoreType.DMA,
        ],
        compiler_params=pltpu.CompilerParams(use_tc_tiling_on_sc=False),
    )(tokens_flat, token_table, pos_embed)
    return out.reshape(BATCH, NTOKEN, NEMBED)
